# baseline (speedup 1.0000x reference)
import functools

import jax
import jax.numpy as jnp
import numpy as np
from jax import lax
from jax.experimental import pallas as pl
from jax.experimental.pallas import tpu as pltpu
from jax.experimental.pallas import tpu_sc as plsc

EPS = 1e-12
H = W = 224
HW = H * W
TOP_K = 2048
NMS_DIST = 4
BORDER = 4
THRESH = 0.0005
TAPS = [(dy, dx) for dy in range(3) for dx in range(3)]


def _conv2d(x, w, b):
    y = lax.conv_general_dilated(x, w, (1, 1), 'SAME',
                                 dimension_numbers=('NCHW', 'OIHW', 'NCHW'))
    return y + b[None, :, None, None]


def _batchnorm(x, p):
    return (p['g'][None, :, None, None] * (x - p['mean'][None, :, None, None])
            / jnp.sqrt(p['var'][None, :, None, None] + 1e-5)
            + p['beta'][None, :, None, None])


def _vgg_block(x, p):
    return jax.nn.relu(_batchnorm(_conv2d(x, p['w'], p['b']), p))


def _nms_body(s_ref, o_ref):
    x = s_ref[0]
    neg = jnp.float32(-jnp.inf)
    pad_w = jnp.full((H, 4), neg, jnp.float32)
    xp = jnp.concatenate([pad_w, x, pad_w], axis=1)
    m = xp[:, 0:W]
    for d in range(1, 9):
        m = jnp.maximum(m, xp[:, d:d + W])
    pad_h = jnp.full((4, W), neg, jnp.float32)
    mp = jnp.concatenate([pad_h, m, pad_h], axis=0)
    p = mp[0:H, :]
    for d in range(1, 9):
        p = jnp.maximum(p, mp[d:d + H, :])
    rows = lax.broadcasted_iota(jnp.int32, (H, W), 0)
    cols = lax.broadcasted_iota(jnp.int32, (H, W), 1)
    border = ((rows >= BORDER) & (rows < H - BORDER)
              & (cols >= BORDER) & (cols < W - BORDER))
    keep = (x == p) & (x > THRESH) & border
    o_ref[0] = jnp.where(keep, x, 0.0)


def _nms_flat(score):
    out = pl.pallas_call(
        _nms_body,
        grid=(score.shape[0],),
        in_specs=[pl.BlockSpec((1, H, W), lambda b: (b, 0, 0))],
        out_specs=pl.BlockSpec((1, H, W), lambda b: (b, 0, 0)),
        out_shape=jax.ShapeDtypeStruct(score.shape, jnp.float32),
    )(score)
    return out.reshape(score.shape[0], HW)


def _conv_body(relu, cout, x0, x1, x2, w_ref, b_ref, o_ref):
    xs = [x0, x1, x2]
    a = jnp.concatenate([xs[dy][0, :, dx:dx + W, :] for dy, dx in TAPS], axis=2)
    acc = lax.dot_general(a, w_ref[...], (((2,), (0,)), ((), ())),
                          preferred_element_type=jnp.float32)
    acc = acc + b_ref[...].astype(jnp.float32)
    if relu:
        acc = jnp.maximum(acc, 0.0)
    o_ref[0] = acc.astype(o_ref.dtype)


def _conv3x3(x_bf, w_cat, bias, relu, out_dtype, bh=8):
    B, _, _, cin = x_bf.shape
    cout = w_cat.shape[1]
    xp = jnp.pad(x_bf, ((0, 0), (1, 1), (1, 1), (0, 0)))
    shifted = [xp[:, i:i + H] for i in range(3)]
    body = functools.partial(_conv_body, relu, cout)
    return pl.pallas_call(
        body,
        grid=(B, H // bh),
        in_specs=[pl.BlockSpec((1, bh, W + 2, cin), lambda b, i: (b, i, 0, 0))] * 3
        + [pl.BlockSpec((9 * cin, cout), lambda b, i: (0, 0)),
           pl.BlockSpec((cout,), lambda b, i: (0,))],
        out_specs=pl.BlockSpec((1, bh, W, cout), lambda b, i: (b, i, 0, 0)),
        out_shape=jax.ShapeDtypeStruct((B, H, W, cout), out_dtype),
    )(*shifted, w_cat, bias)


def _desc_norm_body(d_ref, o_ref):
    d = d_ref[0].astype(jnp.float32)
    n = jnp.sqrt(jnp.sum(d * d, axis=-1, keepdims=True) + EPS)
    o_ref[0] = d / n


def _normalize_rows(d, bh):
    B, M, C = d.shape
    return pl.pallas_call(
        _desc_norm_body,
        grid=(B, M // bh),
        in_specs=[pl.BlockSpec((1, bh, C), lambda b, i: (b, i, 0))],
        out_specs=pl.BlockSpec((1, bh, C), lambda b, i: (b, i, 0)),
        out_shape=jax.ShapeDtypeStruct((B, M, C), jnp.float32),
    )(d)


_NC, _NS = 2, 16
_NW = _NC * _NS


def _sc_gather(table, idx):
    n = idx.shape[0]
    per_w = n // _NW
    mesh = plsc.VectorSubcoreMesh(core_axis_name="c", subcore_axis_name="s")

    @functools.partial(
        pl.kernel, mesh=mesh,
        out_type=jax.ShapeDtypeStruct((n, 128), jnp.float32),
        scratch_types=[
            pltpu.VMEM((per_w,), jnp.int32),
            pltpu.VMEM((per_w, 128), jnp.float32),
            pltpu.SemaphoreType.DMA,
        ],
    )
    def gk(table_hbm, idx_hbm, out_hbm, idx_v, rows_v, sem):
        wid = lax.axis_index("s") * _NC + lax.axis_index("c")
        base = wid * per_w
        pltpu.sync_copy(idx_hbm.at[pl.ds(base, per_w)], idx_v)
        pltpu.async_copy(table_hbm.at[idx_v], rows_v, sem).wait()
        pltpu.sync_copy(rows_v, out_hbm.at[pl.ds(base, per_w)])

    return gk(table, idx)


def kernel(image, params):
    B = image.shape[0]

    x = image / 255.0
    h = x
    for p in params['backbone']:
        h = _vgg_block(h, p)
    features = h
    t = _vgg_block(features, params['det'][0])
    logits = _conv2d(t, params['det'][1]['w'], params['det'][1]['b'])
    score = jax.nn.sigmoid(logits)[:, 0]

    flat = _nms_flat(score)
    vals, idx = lax.top_k(flat, TOP_K)
    kth = vals[:, -1]
    nms = jnp.where((flat >= kth[:, None]) & (flat > 0.0), flat, 0.0).reshape(B, H, W)
    ys = (idx // W).astype(jnp.float32)
    xs = (idx % W).astype(jnp.float32)
    sparse_positions = jnp.stack([ys, xs, vals], axis=-1)

    scale = params['scale']
    feat_hwc = jnp.transpose(features, (0, 2, 3, 1)).astype(jnp.bfloat16)
    d0 = params['desc'][0]
    bnscale = d0['g'] / jnp.sqrt(d0['var'] + 1e-5)
    w0 = jnp.concatenate(
        [d0['w'][:, :, dy, dx] for dy, dx in TAPS], axis=1).T * bnscale[None, :]
    b0 = (d0['b'] - d0['mean']) * bnscale + d0['beta']
    t2 = _conv3x3(feat_hwc, w0.astype(jnp.bfloat16), b0, True, jnp.bfloat16)

    d1 = params['desc'][1]
    w1 = jnp.concatenate([d1['w'][:, :, dy, dx] for dy, dx in TAPS], axis=1).T
    raw_hwc = _conv3x3(t2, w1.astype(jnp.bfloat16), d1['b'], False, jnp.float32)

    raw_rows = raw_hwc.reshape(B, HW, 128)
    norm_rows = scale * _normalize_rows(raw_rows, 224)
    normalized_descriptors = jnp.transpose(
        norm_rows.reshape(B, H, W, 128), (0, 3, 1, 2))
    dense_descriptors = norm_rows

    gidx = (idx + (jnp.arange(B, dtype=jnp.int32) * HW)[:, None]).reshape(-1)
    g = _sc_gather(raw_rows.reshape(B * HW, 128), gidx).reshape(B, TOP_K, 128)
    sparse_descriptors = scale * _normalize_rows(g, 256)

    yy, xx = np.meshgrid(np.arange(H, dtype=np.float32),
                         np.arange(W, dtype=np.float32), indexing='ij')
    coords = jnp.asarray(np.stack([yy.reshape(-1), xx.reshape(-1)], axis=-1))
    dense_positions = jnp.concatenate(
        [jnp.broadcast_to(coords[None], (B, HW, 2)), score.reshape(B, HW, 1)],
        axis=-1)

    return (score, nms, normalized_descriptors, sparse_positions,
            sparse_descriptors, dense_positions, dense_descriptors)


# speedup vs baseline: 1.1555x; 1.1555x over previous
import functools

import jax
import jax.numpy as jnp
import numpy as np
from jax import lax
from jax.experimental import pallas as pl
from jax.experimental.pallas import tpu as pltpu
from jax.experimental.pallas import tpu_sc as plsc

EPS = 1e-12
H = W = 224
HW = H * W
TOP_K = 2048
NMS_DIST = 4
BORDER = 4
THRESH = 0.0005
TAPS = [(dy, dx) for dy in range(3) for dx in range(3)]


def _conv2d(x, w, b):
    y = lax.conv_general_dilated(x, w, (1, 1), 'SAME',
                                 dimension_numbers=('NCHW', 'OIHW', 'NCHW'))
    return y + b[None, :, None, None]


def _batchnorm(x, p):
    return (p['g'][None, :, None, None] * (x - p['mean'][None, :, None, None])
            / jnp.sqrt(p['var'][None, :, None, None] + 1e-5)
            + p['beta'][None, :, None, None])


def _vgg_block(x, p):
    return jax.nn.relu(_batchnorm(_conv2d(x, p['w'], p['b']), p))


def _nms_body(s_ref, o_ref):
    x = s_ref[0]
    neg = jnp.float32(-jnp.inf)
    pad_w = jnp.full((H, 4), neg, jnp.float32)
    xp = jnp.concatenate([pad_w, x, pad_w], axis=1)
    m = xp[:, 0:W]
    for d in range(1, 9):
        m = jnp.maximum(m, xp[:, d:d + W])
    pad_h = jnp.full((4, W), neg, jnp.float32)
    mp = jnp.concatenate([pad_h, m, pad_h], axis=0)
    p = mp[0:H, :]
    for d in range(1, 9):
        p = jnp.maximum(p, mp[d:d + H, :])
    rows = lax.broadcasted_iota(jnp.int32, (H, W), 0)
    cols = lax.broadcasted_iota(jnp.int32, (H, W), 1)
    border = ((rows >= BORDER) & (rows < H - BORDER)
              & (cols >= BORDER) & (cols < W - BORDER))
    keep = (x == p) & (x > THRESH) & border
    o_ref[0] = jnp.where(keep, x, 0.0)


def _nms_flat(score):
    out = pl.pallas_call(
        _nms_body,
        grid=(score.shape[0],),
        in_specs=[pl.BlockSpec((1, H, W), lambda b: (b, 0, 0))],
        out_specs=pl.BlockSpec((1, H, W), lambda b: (b, 0, 0)),
        out_shape=jax.ShapeDtypeStruct(score.shape, jnp.float32),
    )(score)
    return out.reshape(score.shape[0], HW)


def _conv_body(mode, bh, x_ref, w_ref, b_ref, o_ref):
    xb = x_ref[0]
    a = jnp.concatenate(
        [xb[dy:dy + bh, dx:dx + W, :] for dy, dx in TAPS], axis=2)
    acc = lax.dot_general(a, w_ref[...], (((2,), (0,)), ((), ())),
                          preferred_element_type=jnp.float32)
    acc = acc + b_ref[...].astype(jnp.float32)
    if mode == "relu":
        o_ref[0] = jnp.maximum(acc, 0.0).astype(o_ref.dtype)
    elif mode == "norm":
        n = jnp.sqrt(jnp.sum(acc * acc, axis=-1, keepdims=True) + EPS)
        o_ref[0] = (acc / n).reshape(bh * W, 128)
    else:
        o_ref[0] = acc.astype(o_ref.dtype)


def _conv3x3(x_bf, w_cat, bias, mode, out_dtype, bh=8):
    B, _, _, cin = x_bf.shape
    cout = w_cat.shape[1]
    nblk = H // bh
    xp = jnp.pad(x_bf, ((0, 0), (1, 1), (1, 1), (0, 0)))
    bands = jnp.concatenate(
        [xp[:, i * bh:i * bh + bh + 2] for i in range(nblk)], axis=0)
    body = functools.partial(_conv_body, mode, bh)
    if mode == "norm":
        out_specs = pl.BlockSpec((1, bh * W, cout), lambda b, i: (b, i, 0))
        out_shape = jax.ShapeDtypeStruct((B, HW, cout), out_dtype)
    else:
        out_specs = pl.BlockSpec((1, bh, W, cout), lambda b, i: (b, i, 0, 0))
        out_shape = jax.ShapeDtypeStruct((B, H, W, cout), out_dtype)
    return pl.pallas_call(
        body,
        grid=(B, nblk),
        in_specs=[pl.BlockSpec((1, bh + 2, W + 2, cin), lambda b, i: (i * B + b, 0, 0, 0)),
                  pl.BlockSpec((9 * cin, cout), lambda b, i: (0, 0)),
                  pl.BlockSpec((cout,), lambda b, i: (0,))],
        out_specs=out_specs,
        out_shape=out_shape,
    )(bands, w_cat, bias)


def _desc_norm_body(d_ref, o_ref):
    d = d_ref[0].astype(jnp.float32)
    n = jnp.sqrt(jnp.sum(d * d, axis=-1, keepdims=True) + EPS)
    o_ref[0] = d / n


def _normalize_rows(d, bh):
    B, M, C = d.shape
    return pl.pallas_call(
        _desc_norm_body,
        grid=(B, M // bh),
        in_specs=[pl.BlockSpec((1, bh, C), lambda b, i: (b, i, 0))],
        out_specs=pl.BlockSpec((1, bh, C), lambda b, i: (b, i, 0)),
        out_shape=jax.ShapeDtypeStruct((B, M, C), jnp.float32),
    )(d)


_NC, _NS = 2, 16
_NW = _NC * _NS


def _sc_gather(table, idx):
    n = idx.shape[0]
    per_w = n // _NW
    mesh = plsc.VectorSubcoreMesh(core_axis_name="c", subcore_axis_name="s")

    @functools.partial(
        pl.kernel, mesh=mesh,
        out_type=jax.ShapeDtypeStruct((n, 128), jnp.float32),
        scratch_types=[
            pltpu.VMEM((per_w,), jnp.int32),
            pltpu.VMEM((per_w, 128), jnp.float32),
            pltpu.SemaphoreType.DMA,
        ],
    )
    def gk(table_hbm, idx_hbm, out_hbm, idx_v, rows_v, sem):
        wid = lax.axis_index("s") * _NC + lax.axis_index("c")
        base = wid * per_w
        pltpu.sync_copy(idx_hbm.at[pl.ds(base, per_w)], idx_v)
        pltpu.async_copy(table_hbm.at[idx_v], rows_v, sem).wait()
        pltpu.sync_copy(rows_v, out_hbm.at[pl.ds(base, per_w)])

    return gk(table, idx)


def kernel(image, params):
    B = image.shape[0]

    x = image / 255.0
    h = x
    for p in params['backbone']:
        h = _vgg_block(h, p)
    features = h
    t = _vgg_block(features, params['det'][0])
    logits = _conv2d(t, params['det'][1]['w'], params['det'][1]['b'])
    score = jax.nn.sigmoid(logits)[:, 0]

    flat = _nms_flat(score)
    vals, idx = lax.top_k(flat, TOP_K)
    kth = vals[:, -1]
    nms = jnp.where((flat >= kth[:, None]) & (flat > 0.0), flat, 0.0).reshape(B, H, W)
    ys = (idx // W).astype(jnp.float32)
    xs = (idx % W).astype(jnp.float32)
    sparse_positions = jnp.stack([ys, xs, vals], axis=-1)

    scale = params['scale']
    feat_hwc = jnp.transpose(features, (0, 2, 3, 1)).astype(jnp.bfloat16)
    d0 = params['desc'][0]
    bnscale = d0['g'] / jnp.sqrt(d0['var'] + 1e-5)
    w0 = jnp.concatenate(
        [d0['w'][:, :, dy, dx] for dy, dx in TAPS], axis=1).T * bnscale[None, :]
    b0 = (d0['b'] - d0['mean']) * bnscale + d0['beta']
    t2 = _conv3x3(feat_hwc, w0.astype(jnp.bfloat16), b0, "relu", jnp.bfloat16)

    d1 = params['desc'][1]
    w1 = jnp.concatenate([d1['w'][:, :, dy, dx] for dy, dx in TAPS], axis=1).T
    unit_rows = _conv3x3(t2, w1.astype(jnp.bfloat16), d1['b'], "norm", jnp.float32)
    norm_rows = scale * unit_rows
    normalized_descriptors = jnp.transpose(
        norm_rows.reshape(B, H, W, 128), (0, 3, 1, 2))
    dense_descriptors = norm_rows

    gidx = (idx + (jnp.arange(B, dtype=jnp.int32) * HW)[:, None]).reshape(-1)
    g = _sc_gather(unit_rows.reshape(B * HW, 128), gidx).reshape(B, TOP_K, 128)
    sparse_descriptors = scale * _normalize_rows(g, 256)

    yy, xx = np.meshgrid(np.arange(H, dtype=np.float32),
                         np.arange(W, dtype=np.float32), indexing='ij')
    coords = jnp.asarray(np.stack([yy.reshape(-1), xx.reshape(-1)], axis=-1))
    dense_positions = jnp.concatenate(
        [jnp.broadcast_to(coords[None], (B, HW, 2)), score.reshape(B, HW, 1)],
        axis=-1)

    return (score, nms, normalized_descriptors, sparse_positions,
            sparse_descriptors, dense_positions, dense_descriptors)


# speedup vs baseline: 1.1754x; 1.0172x over previous
import functools

import jax
import jax.numpy as jnp
import numpy as np
from jax import lax
from jax.experimental import pallas as pl
from jax.experimental.pallas import tpu as pltpu
from jax.experimental.pallas import tpu_sc as plsc

EPS = 1e-12
H = W = 224
HW = H * W
TOP_K = 2048
NMS_DIST = 4
BORDER = 4
THRESH = 0.0005
TAPS = [(dy, dx) for dy in range(3) for dx in range(3)]


def _conv2d(x, w, b):
    y = lax.conv_general_dilated(x, w, (1, 1), 'SAME',
                                 dimension_numbers=('NCHW', 'OIHW', 'NCHW'))
    return y + b[None, :, None, None]


def _batchnorm(x, p):
    return (p['g'][None, :, None, None] * (x - p['mean'][None, :, None, None])
            / jnp.sqrt(p['var'][None, :, None, None] + 1e-5)
            + p['beta'][None, :, None, None])


def _vgg_block(x, p):
    return jax.nn.relu(_batchnorm(_conv2d(x, p['w'], p['b']), p))


def _nms_body(s_ref, o_ref):
    x = s_ref[0]
    neg = jnp.float32(-jnp.inf)
    pad_w = jnp.full((H, 4), neg, jnp.float32)
    xp = jnp.concatenate([pad_w, x, pad_w], axis=1)
    m = xp[:, 0:W]
    for d in range(1, 9):
        m = jnp.maximum(m, xp[:, d:d + W])
    pad_h = jnp.full((4, W), neg, jnp.float32)
    mp = jnp.concatenate([pad_h, m, pad_h], axis=0)
    p = mp[0:H, :]
    for d in range(1, 9):
        p = jnp.maximum(p, mp[d:d + H, :])
    rows = lax.broadcasted_iota(jnp.int32, (H, W), 0)
    cols = lax.broadcasted_iota(jnp.int32, (H, W), 1)
    border = ((rows >= BORDER) & (rows < H - BORDER)
              & (cols >= BORDER) & (cols < W - BORDER))
    keep = (x == p) & (x > THRESH) & border
    o_ref[0] = jnp.where(keep, x, 0.0)


def _nms_flat(score):
    out = pl.pallas_call(
        _nms_body,
        grid=(score.shape[0],),
        in_specs=[pl.BlockSpec((1, H, W), lambda b: (b, 0, 0))],
        out_specs=pl.BlockSpec((1, H, W), lambda b: (b, 0, 0)),
        out_shape=jax.ShapeDtypeStruct(score.shape, jnp.float32),
    )(score)
    return out.reshape(score.shape[0], HW)


def _conv_body(mode, bh, x_ref, w_ref, b_ref, s_ref, o_ref):
    xb = x_ref[0]
    a = jnp.concatenate(
        [xb[dy:dy + bh, dx:dx + W, :] for dy, dx in TAPS], axis=2)
    acc = lax.dot_general(a, w_ref[...], (((2,), (0,)), ((), ())),
                          preferred_element_type=jnp.float32)
    acc = acc + b_ref[...].astype(jnp.float32)
    if mode == "relu":
        o_ref[0] = jnp.maximum(acc, 0.0).astype(o_ref.dtype)
    elif mode == "norm":
        n = jnp.sqrt(jnp.sum(acc * acc, axis=-1, keepdims=True) + EPS)
        o_ref[0] = (s_ref[0] * (acc / n)).reshape(bh * W, 128)
    else:
        o_ref[0] = acc.astype(o_ref.dtype)


def _conv3x3(x_bf, w_cat, bias, mode, out_dtype, scale_arr, bh=8):
    B, _, _, cin = x_bf.shape
    cout = w_cat.shape[1]
    nblk = H // bh
    xp = jnp.pad(x_bf, ((0, 0), (1, 1), (1, 1), (0, 0)))
    bands = jnp.concatenate(
        [xp[:, i * bh:i * bh + bh + 2] for i in range(nblk)], axis=0)
    body = functools.partial(_conv_body, mode, bh)
    if mode == "norm":
        out_specs = pl.BlockSpec((1, bh * W, cout), lambda b, i: (b, i, 0))
        out_shape = jax.ShapeDtypeStruct((B, HW, cout), out_dtype)
    else:
        out_specs = pl.BlockSpec((1, bh, W, cout), lambda b, i: (b, i, 0, 0))
        out_shape = jax.ShapeDtypeStruct((B, H, W, cout), out_dtype)
    return pl.pallas_call(
        body,
        grid=(B, nblk),
        in_specs=[pl.BlockSpec((1, bh + 2, W + 2, cin), lambda b, i: (i * B + b, 0, 0, 0)),
                  pl.BlockSpec((9 * cin, cout), lambda b, i: (0, 0)),
                  pl.BlockSpec((cout,), lambda b, i: (0,)),
                  pl.BlockSpec((1,), lambda b, i: (0,))],
        out_specs=out_specs,
        out_shape=out_shape,
    )(bands, w_cat, bias, scale_arr)


def _desc_norm_body(d_ref, o_ref):
    d = d_ref[0].astype(jnp.float32)
    n = jnp.sqrt(jnp.sum(d * d, axis=-1, keepdims=True) + EPS)
    o_ref[0] = d / n


def _normalize_rows(d, bh):
    B, M, C = d.shape
    return pl.pallas_call(
        _desc_norm_body,
        grid=(B, M // bh),
        in_specs=[pl.BlockSpec((1, bh, C), lambda b, i: (b, i, 0))],
        out_specs=pl.BlockSpec((1, bh, C), lambda b, i: (b, i, 0)),
        out_shape=jax.ShapeDtypeStruct((B, M, C), jnp.float32),
    )(d)


_NC, _NS = 2, 16
_NW = _NC * _NS


def _sc_gather(table, idx):
    n = idx.shape[0]
    per_w = n // _NW
    mesh = plsc.VectorSubcoreMesh(core_axis_name="c", subcore_axis_name="s")

    @functools.partial(
        pl.kernel, mesh=mesh,
        out_type=jax.ShapeDtypeStruct((n, 128), jnp.float32),
        scratch_types=[
            pltpu.VMEM((per_w,), jnp.int32),
            pltpu.VMEM((per_w, 128), jnp.float32),
            pltpu.SemaphoreType.DMA,
        ],
    )
    def gk(table_hbm, idx_hbm, out_hbm, idx_v, rows_v, sem):
        wid = lax.axis_index("s") * _NC + lax.axis_index("c")
        base = wid * per_w
        pltpu.sync_copy(idx_hbm.at[pl.ds(base, per_w)], idx_v)
        pltpu.async_copy(table_hbm.at[idx_v], rows_v, sem).wait()
        pltpu.sync_copy(rows_v, out_hbm.at[pl.ds(base, per_w)])

    return gk(table, idx)


def kernel(image, params):
    B = image.shape[0]

    x = image / 255.0
    h = x
    for p in params['backbone']:
        h = _vgg_block(h, p)
    features = h
    t = _vgg_block(features, params['det'][0])
    logits = _conv2d(t, params['det'][1]['w'], params['det'][1]['b'])
    score = jax.nn.sigmoid(logits)[:, 0]

    flat = _nms_flat(score)
    vals, idx = lax.top_k(flat, TOP_K)
    kth = vals[:, -1]
    nms = jnp.where((flat >= kth[:, None]) & (flat > 0.0), flat, 0.0).reshape(B, H, W)
    ys = (idx // W).astype(jnp.float32)
    xs = (idx % W).astype(jnp.float32)
    sparse_positions = jnp.stack([ys, xs, vals], axis=-1)

    scale = params['scale']
    feat_hwc = jnp.transpose(features, (0, 2, 3, 1)).astype(jnp.bfloat16)
    d0 = params['desc'][0]
    bnscale = d0['g'] / jnp.sqrt(d0['var'] + 1e-5)
    w0 = jnp.concatenate(
        [d0['w'][:, :, dy, dx] for dy, dx in TAPS], axis=1).T * bnscale[None, :]
    b0 = (d0['b'] - d0['mean']) * bnscale + d0['beta']
    one = jnp.ones((1,), jnp.float32)
    t2 = _conv3x3(feat_hwc, w0.astype(jnp.bfloat16), b0, "relu", jnp.bfloat16, one)

    d1 = params['desc'][1]
    w1 = jnp.concatenate([d1['w'][:, :, dy, dx] for dy, dx in TAPS], axis=1).T
    norm_rows = _conv3x3(t2, w1.astype(jnp.bfloat16), d1['b'], "norm",
                         jnp.float32, scale.reshape(1))
    normalized_descriptors = jnp.transpose(
        norm_rows.reshape(B, H, W, 128), (0, 3, 1, 2))
    dense_descriptors = norm_rows

    gidx = (idx + (jnp.arange(B, dtype=jnp.int32) * HW)[:, None]).reshape(-1)
    g = _sc_gather(norm_rows.reshape(B * HW, 128), gidx).reshape(B, TOP_K, 128)
    sparse_descriptors = scale * _normalize_rows(g, 256)

    yy, xx = np.meshgrid(np.arange(H, dtype=np.float32),
                         np.arange(W, dtype=np.float32), indexing='ij')
    coords = jnp.asarray(np.stack([yy.reshape(-1), xx.reshape(-1)], axis=-1))
    dense_positions = jnp.concatenate(
        [jnp.broadcast_to(coords[None], (B, HW, 2)), score.reshape(B, HW, 1)],
        axis=-1)

    return (score, nms, normalized_descriptors, sparse_positions,
            sparse_descriptors, dense_positions, dense_descriptors)
